# baseline (speedup 1.0000x reference)
import functools
import jax
import jax.numpy as jnp
from jax import lax
from jax.experimental import pallas as pl
from jax.experimental.pallas import tpu as pltpu, tpu_sc as plsc

_N = 1020
_NNZ = 16320
_B = 16
_NPAD = 1024
_EPAD = 16384
_NC = 2
_NS = 16
_EPT = _EPAD // _NS
_GROUPS = _EPT // 16
_ROWS_PER_CORE = _NPAD // _NC
_ROWS_PER_TILE = _ROWS_PER_CORE // _NS
_ACC = _ROWS_PER_CORE * _NPAD
_TILE_SLICE = _ROWS_PER_TILE * _NPAD


def _densify_body(rows_hbm, cols_hbm, vals_hbm, zeros_hbm, out_hbm,
                  rows_v, cols_v, vals_v, idx_v, upd_v, acc0, acc1,
                  sem_e, sem_z, sem_s, sem_o):
    cid = lax.axis_index("c")
    sid = lax.axis_index("s")
    lo = cid * _ROWS_PER_CORE
    hi = lo + _ROWS_PER_CORE
    base_e = sid * _EPT
    my = pl.ds(sid * _TILE_SLICE, _TILE_SLICE)
    row0 = (cid * _ROWS_PER_CORE + sid * _ROWS_PER_TILE) * _NPAD
    accs = (acc0, acc1)

    stages = [
        pltpu.async_copy(rows_hbm.at[:, pl.ds(base_e, _EPT)], rows_v, sem_e),
        pltpu.async_copy(cols_hbm.at[:, pl.ds(base_e, _EPT)], cols_v, sem_e),
        pltpu.async_copy(vals_hbm.at[:, pl.ds(base_e, _EPT)], vals_v, sem_e),
    ]
    zs = [pltpu.async_copy(zeros_hbm, acc0.at[my], sem_z),
          pltpu.async_copy(zeros_hbm, acc1.at[my], sem_z)]
    for cp in stages:
        cp.wait()

    for m in range(4):
        for g in range(_GROUPS):
            r = rows_v[m, pl.ds(g * 16, 16)]
            c = cols_v[m, pl.ds(g * 16, 16)]
            v = vals_v[m, pl.ds(g * 16, 16)]
            in_range = (r >= lo) & (r < hi)
            flat = (r - lo) * _NPAD + c
            j, o = g // 8, (g % 8) * 16
            idx_v[m, j, pl.ds(o, 16)] = jnp.where(in_range, flat, 0)
            upd_v[m, j, pl.ds(o, 16)] = jnp.where(in_range, v, 0.0)

    for z in zs:
        z.wait()
    plsc.subcore_barrier()

    ocs = [None] * 4
    rzs = [None, None]
    for m in range(4):
        acc = accs[m % 2]
        if m >= 2:
            rzs[m % 2].wait()
            plsc.subcore_barrier()
        scs = [pltpu.async_copy(upd_v.at[m, j], acc.at[idx_v.at[m, j]],
                                sem_s, add=True)
               for j in range(8)]
        for sc in scs:
            sc.wait()
        plsc.subcore_barrier()
        ocs[m] = pltpu.async_copy(acc.at[my],
                                  out_hbm.at[m, pl.ds(row0, _TILE_SLICE)],
                                  sem_o)
        if m < 2:
            ocs[m].wait()
            rzs[m] = pltpu.async_copy(zeros_hbm, acc.at[my], sem_z)
    ocs[2].wait()
    ocs[3].wait()


@jax.jit
def _densify(rows_all, cols_all, vals_all, zeros32k):
    mesh = plsc.VectorSubcoreMesh(core_axis_name="c", subcore_axis_name="s",
                                  num_cores=_NC, num_subcores=_NS)
    return pl.kernel(
        _densify_body,
        out_type=jax.ShapeDtypeStruct((4, _NPAD * _NPAD), jnp.float32),
        mesh=mesh,
        scratch_types=[
            pltpu.VMEM((4, _EPT), jnp.int32),
            pltpu.VMEM((4, _EPT), jnp.int32),
            pltpu.VMEM((4, _EPT), jnp.float32),
            pltpu.VMEM((4, 8, 128), jnp.int32),
            pltpu.VMEM((4, 8, 128), jnp.float32),
            pltpu.VMEM_SHARED((_ACC,), jnp.float32),
            pltpu.VMEM_SHARED((_ACC,), jnp.float32),
            pltpu.SemaphoreType.DMA,
            pltpu.SemaphoreType.DMA,
            pltpu.SemaphoreType.DMA,
            pltpu.SemaphoreType.DMA,
        ],
    )(rows_all, cols_all, vals_all, zeros32k)


_A_OF_LAYER = (0, 0, 1, 2, 3, 3)

_BPG = 8


def _forward_body(h_ref, a_ref, w0, b0, w1, b1, w2, b2, w3, b3, w4, b4, w5, b5,
                  o_ref):
    ws = (w0, w1, w2, w3, w4, w5)
    bs = (b0, b1, b2, b3, b4, b5)
    xs = [h_ref[k] for k in range(_BPG)]
    for i in range(6):
        for k in range(_BPG):
            h = (jnp.dot(xs[k], ws[i][...], preferred_element_type=jnp.float32)
                 + bs[i][...])
            y = jnp.dot(a_ref[_A_OF_LAYER[i]], h,
                        preferred_element_type=jnp.float32)
            xs[k] = jnp.maximum(y, 0.0)
    for k in range(_BPG):
        o_ref[k] = xs[k]


@jax.jit
def _forward(Hp, As, W0, b0, W1, b1, W2, b2, W3, b3, W4, b4, W5, b5):
    full = lambda arr: pl.BlockSpec(arr.shape, lambda b: (0,) * arr.ndim)
    weight_specs = [full(w) for w in
                    (W0, b0, W1, b1, W2, b2, W3, b3, W4, b4, W5, b5)]
    return pl.pallas_call(
        _forward_body,
        grid=(_B // _BPG,),
        in_specs=[
            pl.BlockSpec((_BPG, _NPAD, 2), lambda b: (b, 0, 0)),
            pl.BlockSpec((4, _NPAD, _NPAD), lambda b: (0, 0, 0)),
            *weight_specs,
        ],
        out_specs=pl.BlockSpec((_BPG, _NPAD, 2), lambda b: (b, 0, 0)),
        out_shape=jax.ShapeDtypeStruct((_B, _NPAD, 2), jnp.float32),
        compiler_params=pltpu.CompilerParams(
            dimension_semantics=("arbitrary",),
            vmem_limit_bytes=100 * 1024 * 1024,
        ),
    )(Hp, As, W0, b0, W1, b1, W2, b2, W3, b3, W4, b4, W5, b5)


def kernel(H, s_sm_rows, s_sm_cols, s_sm_vals, s_sp_rows, s_sp_cols, s_sp_vals,
           t_sm_rows, t_sm_cols, t_sm_vals, t_sp_rows, t_sp_cols, t_sp_vals,
           W0, b0, W1, b1, W2, b2, W3, b3, W4, b4, W5, b5):
    pad_e = _EPAD - _NNZ
    rows_all = jnp.stack([jnp.pad(r, (0, pad_e)) for r in
                          (s_sm_rows, t_sm_rows, t_sp_rows, s_sp_rows)])
    cols_all = jnp.stack([jnp.pad(c, (0, pad_e)) for c in
                          (s_sm_cols, t_sm_cols, t_sp_cols, s_sp_cols)])
    vals_all = jnp.stack([jnp.pad(v, (0, pad_e)) for v in
                          (s_sm_vals, t_sm_vals, t_sp_vals, s_sp_vals)])
    zeros32k = jnp.zeros((_TILE_SLICE,), jnp.float32)

    As = _densify(rows_all, cols_all, vals_all, zeros32k).reshape(4, _NPAD, _NPAD)

    Hp = jnp.pad(H, ((0, 0), (0, _NPAD - _N), (0, 0)))
    bias = lambda b: b.reshape(1, -1)
    out = _forward(Hp, As, W0, bias(b0), W1, bias(b1), W2, bias(b2),
                   W3, bias(b3), W4, bias(b4), W5, bias(b5))
    return out[:, :_N, :]


# speedup vs baseline: 1.1450x; 1.1450x over previous
import functools
import jax
import jax.numpy as jnp
from jax import lax
from jax.experimental import pallas as pl
from jax.experimental.pallas import tpu as pltpu, tpu_sc as plsc

_N = 1020
_NNZ = 16320
_B = 16
_NPAD = 1024
_EPAD = 16384
_NC = 2
_NS = 16
_EPT = _EPAD // _NS
_GROUPS = _EPT // 16
_ROWS_PER_CORE = _NPAD // _NC
_ROWS_PER_TILE = _ROWS_PER_CORE // _NS
_ACC = _ROWS_PER_CORE * _NPAD
_TILE_SLICE = _ROWS_PER_TILE * _NPAD


def _densify_body(rows_hbm, cols_hbm, vals_hbm, zeros_hbm, out_hbm,
                  rows_v, cols_v, vals_v, idx_v, upd_v, acc0, acc1,
                  sem_e, sem_z, sem_s, sem_o):
    cid = lax.axis_index("c")
    sid = lax.axis_index("s")
    lo = cid * _ROWS_PER_CORE
    hi = lo + _ROWS_PER_CORE
    base_e = sid * _EPT
    my = pl.ds(sid * _TILE_SLICE, _TILE_SLICE)
    row0 = (cid * _ROWS_PER_CORE + sid * _ROWS_PER_TILE) * _NPAD
    accs = (acc0, acc1)

    stages = [
        pltpu.async_copy(rows_hbm.at[:, pl.ds(base_e, _EPT)], rows_v, sem_e),
        pltpu.async_copy(cols_hbm.at[:, pl.ds(base_e, _EPT)], cols_v, sem_e),
        pltpu.async_copy(vals_hbm.at[:, pl.ds(base_e, _EPT)], vals_v, sem_e),
    ]
    zs = [pltpu.async_copy(zeros_hbm, acc0.at[my], sem_z),
          pltpu.async_copy(zeros_hbm, acc1.at[my], sem_z)]
    for cp in stages:
        cp.wait()

    for m in range(4):
        for g in range(_GROUPS):
            r = rows_v[m, pl.ds(g * 16, 16)]
            c = cols_v[m, pl.ds(g * 16, 16)]
            v = vals_v[m, pl.ds(g * 16, 16)]
            in_range = (r >= lo) & (r < hi)
            flat = (r - lo) * _NPAD + c
            j, o = g // 8, (g % 8) * 16
            idx_v[m, j, pl.ds(o, 16)] = jnp.where(in_range, flat, 0)
            upd_v[m, j, pl.ds(o, 16)] = jnp.where(in_range, v, 0.0)

    for z in zs:
        z.wait()
    plsc.subcore_barrier()

    ocs = [None] * 4
    rzs = [None, None]
    for m in range(4):
        acc = accs[m % 2]
        if m >= 2:
            rzs[m % 2].wait()
            plsc.subcore_barrier()
        scs = [pltpu.async_copy(upd_v.at[m, j], acc.at[idx_v.at[m, j]],
                                sem_s, add=True)
               for j in range(8)]
        for sc in scs:
            sc.wait()
        plsc.subcore_barrier()
        ocs[m] = pltpu.async_copy(acc.at[my],
                                  out_hbm.at[m, pl.ds(row0, _TILE_SLICE)],
                                  sem_o)
        if m < 2:
            ocs[m].wait()
            rzs[m] = pltpu.async_copy(zeros_hbm, acc.at[my], sem_z)
    ocs[2].wait()
    ocs[3].wait()


@jax.jit
def _densify(rows_all, cols_all, vals_all, zeros32k):
    mesh = plsc.VectorSubcoreMesh(core_axis_name="c", subcore_axis_name="s",
                                  num_cores=_NC, num_subcores=_NS)
    return pl.kernel(
        _densify_body,
        out_type=jax.ShapeDtypeStruct((4, _NPAD * _NPAD), jnp.float32),
        mesh=mesh,
        scratch_types=[
            pltpu.VMEM((4, _EPT), jnp.int32),
            pltpu.VMEM((4, _EPT), jnp.int32),
            pltpu.VMEM((4, _EPT), jnp.float32),
            pltpu.VMEM((4, 8, 128), jnp.int32),
            pltpu.VMEM((4, 8, 128), jnp.float32),
            pltpu.VMEM_SHARED((_ACC,), jnp.float32),
            pltpu.VMEM_SHARED((_ACC,), jnp.float32),
            pltpu.SemaphoreType.DMA,
            pltpu.SemaphoreType.DMA,
            pltpu.SemaphoreType.DMA,
            pltpu.SemaphoreType.DMA,
        ],
    )(rows_all, cols_all, vals_all, zeros32k)


_A_OF_LAYER = (0, 0, 1, 2, 3, 3)

_BPG = 4


def _forward_body(h_ref, a_ref, w0, b0, w1, b1, w2, b2, w3, b3, w4, b4, w5, b5,
                  o_ref):
    ws = (w0, w1, w2, w3, w4, w5)
    bs = (b0, b1, b2, b3, b4, b5)
    xs = [h_ref[k] for k in range(_BPG)]
    for i in range(6):
        for k in range(_BPG):
            h = (jnp.dot(xs[k], ws[i][...], preferred_element_type=jnp.float32)
                 + bs[i][...])
            y = jnp.dot(a_ref[_A_OF_LAYER[i]], h,
                        preferred_element_type=jnp.float32)
            xs[k] = jnp.maximum(y, 0.0)
    for k in range(_BPG):
        o_ref[k] = xs[k]


@jax.jit
def _forward(Hp, As, W0, b0, W1, b1, W2, b2, W3, b3, W4, b4, W5, b5):
    full = lambda arr: pl.BlockSpec(arr.shape, lambda b: (0,) * arr.ndim)
    weight_specs = [full(w) for w in
                    (W0, b0, W1, b1, W2, b2, W3, b3, W4, b4, W5, b5)]
    return pl.pallas_call(
        _forward_body,
        grid=(_B // _BPG,),
        in_specs=[
            pl.BlockSpec((_BPG, _NPAD, 2), lambda b: (b, 0, 0)),
            pl.BlockSpec((4, _NPAD, _NPAD), lambda b: (0, 0, 0)),
            *weight_specs,
        ],
        out_specs=pl.BlockSpec((_BPG, _NPAD, 2), lambda b: (b, 0, 0)),
        out_shape=jax.ShapeDtypeStruct((_B, _NPAD, 2), jnp.float32),
        compiler_params=pltpu.CompilerParams(
            dimension_semantics=("arbitrary",),
            vmem_limit_bytes=100 * 1024 * 1024,
        ),
    )(Hp, As, W0, b0, W1, b1, W2, b2, W3, b3, W4, b4, W5, b5)


def kernel(H, s_sm_rows, s_sm_cols, s_sm_vals, s_sp_rows, s_sp_cols, s_sp_vals,
           t_sm_rows, t_sm_cols, t_sm_vals, t_sp_rows, t_sp_cols, t_sp_vals,
           W0, b0, W1, b1, W2, b2, W3, b3, W4, b4, W5, b5):
    pad_e = _EPAD - _NNZ
    rows_all = jnp.stack([jnp.pad(r, (0, pad_e)) for r in
                          (s_sm_rows, t_sm_rows, t_sp_rows, s_sp_rows)])
    cols_all = jnp.stack([jnp.pad(c, (0, pad_e)) for c in
                          (s_sm_cols, t_sm_cols, t_sp_cols, s_sp_cols)])
    vals_all = jnp.stack([jnp.pad(v, (0, pad_e)) for v in
                          (s_sm_vals, t_sm_vals, t_sp_vals, s_sp_vals)])
    zeros32k = jnp.zeros((_TILE_SLICE,), jnp.float32)

    As = _densify(rows_all, cols_all, vals_all, zeros32k).reshape(4, _NPAD, _NPAD)

    Hp = jnp.pad(H, ((0, 0), (0, _NPAD - _N), (0, 0)))
    bias = lambda b: b.reshape(1, -1)
    out = _forward(Hp, As, W0, bias(b0), W1, bias(b1), W2, bias(b2),
                   W3, bias(b3), W4, bias(b4), W5, bias(b5))
    return out[:, :_N, :]


# speedup vs baseline: 1.3324x; 1.1637x over previous
import functools
import jax
import jax.numpy as jnp
from jax import lax
from jax.experimental import pallas as pl
from jax.experimental.pallas import tpu as pltpu, tpu_sc as plsc

_N = 1020
_NNZ = 16320
_B = 16
_NPAD = 1024
_EPAD = 16384
_NC = 2
_NS = 16
_EPT = _EPAD // _NS
_GROUPS = _EPT // 16
_ROWS_PER_CORE = _NPAD // _NC
_ROWS_PER_TILE = _ROWS_PER_CORE // _NS
_ACC = _ROWS_PER_CORE * _NPAD
_TILE_SLICE = _ROWS_PER_TILE * _NPAD


def _densify_body(nm, rows_hbm, cols_hbm, vals_hbm, zeros_hbm, out_hbm,
                  rows_v, cols_v, vals_v, idx_v, upd_v, acc0, acc1,
                  sem_e, sem_z, sem_s, sem_o):
    cid = lax.axis_index("c")
    sid = lax.axis_index("s")
    lo = cid * _ROWS_PER_CORE
    hi = lo + _ROWS_PER_CORE
    base_e = sid * _EPT
    my = pl.ds(sid * _TILE_SLICE, _TILE_SLICE)
    row0 = (cid * _ROWS_PER_CORE + sid * _ROWS_PER_TILE) * _NPAD
    accs = (acc0, acc1)

    stages = [
        pltpu.async_copy(rows_hbm.at[:, pl.ds(base_e, _EPT)], rows_v, sem_e),
        pltpu.async_copy(cols_hbm.at[:, pl.ds(base_e, _EPT)], cols_v, sem_e),
        pltpu.async_copy(vals_hbm.at[:, pl.ds(base_e, _EPT)], vals_v, sem_e),
    ]
    zs = [pltpu.async_copy(zeros_hbm, accs[p].at[my], sem_z)
          for p in range(min(nm, 2))]
    for cp in stages:
        cp.wait()

    for m in range(nm):
        for g in range(_GROUPS):
            r = rows_v[m, pl.ds(g * 16, 16)]
            c = cols_v[m, pl.ds(g * 16, 16)]
            v = vals_v[m, pl.ds(g * 16, 16)]
            in_range = (r >= lo) & (r < hi)
            flat = (r - lo) * _NPAD + c
            j, o = g // 8, (g % 8) * 16
            idx_v[m, j, pl.ds(o, 16)] = jnp.where(in_range, flat, 0)
            upd_v[m, j, pl.ds(o, 16)] = jnp.where(in_range, v, 0.0)

    for z in zs:
        z.wait()
    plsc.subcore_barrier()

    ocs = [None] * nm
    rzs = [None, None]
    for m in range(nm):
        acc = accs[m % 2]
        if m >= 2:
            rzs[m % 2].wait()
            plsc.subcore_barrier()
        scs = [pltpu.async_copy(upd_v.at[m, j], acc.at[idx_v.at[m, j]],
                                sem_s, add=True)
               for j in range(8)]
        for sc in scs:
            sc.wait()
        plsc.subcore_barrier()
        dst = out_hbm.at[pl.ds(m * _NPAD * _NPAD + row0, _TILE_SLICE)]
        ocs[m] = pltpu.async_copy(acc.at[my], dst, sem_o)
        if m + 2 < nm:
            ocs[m].wait()
            rzs[m % 2] = pltpu.async_copy(zeros_hbm, acc.at[my], sem_z)
    for m in range(max(0, nm - 2), nm):
        ocs[m].wait()


@functools.partial(jax.jit, static_argnums=0)
def _densify(nm, rows_all, cols_all, vals_all, zeros32k):
    mesh = plsc.VectorSubcoreMesh(core_axis_name="c", subcore_axis_name="s",
                                  num_cores=_NC, num_subcores=_NS)
    return pl.kernel(
        functools.partial(_densify_body, nm),
        out_type=jax.ShapeDtypeStruct((nm * _NPAD * _NPAD,), jnp.float32),
        mesh=mesh,
        scratch_types=[
            pltpu.VMEM((nm, _EPT), jnp.int32),
            pltpu.VMEM((nm, _EPT), jnp.int32),
            pltpu.VMEM((nm, _EPT), jnp.float32),
            pltpu.VMEM((nm, 8, 128), jnp.int32),
            pltpu.VMEM((nm, 8, 128), jnp.float32),
            pltpu.VMEM_SHARED((_ACC,), jnp.float32),
            pltpu.VMEM_SHARED((_ACC,), jnp.float32),
            pltpu.SemaphoreType.DMA,
            pltpu.SemaphoreType.DMA,
            pltpu.SemaphoreType.DMA,
            pltpu.SemaphoreType.DMA,
        ],
    )(rows_all, cols_all, vals_all, zeros32k)


_BPG = 4


def _fw_body(a_of_layer, h_ref, a_ref, *refs):
    nl = len(a_of_layer)
    ws = refs[0:2 * nl:2]
    bs = refs[1:2 * nl:2]
    o_ref = refs[2 * nl]
    xs = [h_ref[k] for k in range(_BPG)]
    for i in range(nl):
        for k in range(_BPG):
            h = (jnp.dot(xs[k], ws[i][...], preferred_element_type=jnp.float32)
                 + bs[i][...])
            y = jnp.dot(a_ref[a_of_layer[i]], h,
                        preferred_element_type=jnp.float32)
            xs[k] = jnp.maximum(y, 0.0)
    for k in range(_BPG):
        o_ref[k] = xs[k]


@functools.partial(jax.jit, static_argnums=(0, 1))
def _forward(a_of_layer, dout, Hp, As, *wbs):
    full = lambda arr: pl.BlockSpec(arr.shape, lambda b: (0,) * arr.ndim)
    nmat = As.shape[0]
    din = Hp.shape[-1]
    return pl.pallas_call(
        functools.partial(_fw_body, a_of_layer),
        grid=(_B // _BPG,),
        in_specs=[
            pl.BlockSpec((_BPG, _NPAD, din), lambda b: (b, 0, 0)),
            pl.BlockSpec((nmat, _NPAD, _NPAD), lambda b: (0, 0, 0)),
            *[full(w) for w in wbs],
        ],
        out_specs=pl.BlockSpec((_BPG, _NPAD, dout), lambda b: (b, 0, 0)),
        out_shape=jax.ShapeDtypeStruct((_B, _NPAD, dout), jnp.float32),
        compiler_params=pltpu.CompilerParams(
            dimension_semantics=("arbitrary",),
            vmem_limit_bytes=100 * 1024 * 1024,
        ),
    )(Hp, As, *wbs)


def kernel(H, s_sm_rows, s_sm_cols, s_sm_vals, s_sp_rows, s_sp_cols, s_sp_vals,
           t_sm_rows, t_sm_cols, t_sm_vals, t_sp_rows, t_sp_cols, t_sp_vals,
           W0, b0, W1, b1, W2, b2, W3, b3, W4, b4, W5, b5):
    pad_e = _EPAD - _NNZ
    p = lambda arrs: jnp.stack([jnp.pad(a, (0, pad_e)) for a in arrs])
    zeros32k = jnp.zeros((_TILE_SLICE,), jnp.float32)

    A0 = _densify(1, p([s_sm_rows]), p([s_sm_cols]), p([s_sm_vals]),
                  zeros32k).reshape(1, _NPAD, _NPAD)
    A123 = _densify(3, p([t_sm_rows, t_sp_rows, s_sp_rows]),
                    p([t_sm_cols, t_sp_cols, s_sp_cols]),
                    p([t_sm_vals, t_sp_vals, s_sp_vals]),
                    zeros32k).reshape(3, _NPAD, _NPAD)

    Hp = jnp.pad(H, ((0, 0), (0, _NPAD - _N), (0, 0)))
    bias = lambda b: b.reshape(1, -1)
    x2 = _forward((0, 0), 300, Hp, A0,
                  W0, bias(b0), W1, bias(b1))
    out = _forward((0, 1, 2, 2), 2, x2, A123,
                   W2, bias(b2), W3, bias(b3), W4, bias(b4), W5, bias(b5))
    return out[:, :_N, :]
